# Initial kernel scaffold; baseline (speedup 1.0000x reference)
import jax
import jax.numpy as jnp
from jax.experimental import pallas as pl


def kernel(x, edge_index, W0, b0, W1, b1):
    raise NotImplementedError("write your pallas kernel here")


# speedup vs baseline: 10.2237x; 10.2237x over previous
import functools

import jax
import jax.numpy as jnp
from jax import lax
from jax.experimental import pallas as pl
from jax.experimental.pallas import tpu as pltpu
from jax.experimental.pallas import tpu_sc as plsc

N = 10000
E = 320000
D = 128
NC = 2
NS = 16
NW = NC * NS
EPW = E // NW
CH = 80
NCH = EPW // CH
NP = 10240
RPT = NP // NS
DEG_W = 128

@functools.cache
def _mesh():
    return plsc.VectorSubcoreMesh(core_axis_name="c", subcore_axis_name="s")


def _deg_sc(col, zeros16, ones16):
    @functools.partial(
        pl.kernel, mesh=_mesh(),
        out_type=jax.ShapeDtypeStruct((NC * NP, DEG_W), jnp.float32),
        scratch_types=[
            pltpu.VMEM((CH,), jnp.int32),
            pltpu.VMEM((CH, DEG_W), jnp.float32),
            pltpu.VMEM_SHARED((NP, DEG_W), jnp.float32),
        ])
    def k(col_hbm, z_hbm, ones_hbm, out_hbm, cidx, ones_v, acc):
        c = lax.axis_index("c")
        s = lax.axis_index("s")
        zoff = pl.multiple_of(s * RPT, 8)
        pltpu.sync_copy(z_hbm.at[pl.ds(zoff, RPT)], acc.at[pl.ds(zoff, RPT)])
        pltpu.sync_copy(ones_hbm, ones_v)
        plsc.subcore_barrier()
        base = (c * NS + s) * EPW

        @pl.loop(0, NCH)
        def _(i):
            off = pl.multiple_of(base + i * CH, 8)
            pltpu.sync_copy(col_hbm.at[pl.ds(off, CH)], cidx)
            pltpu.sync_copy(ones_v, acc.at[cidx], add=True)

        plsc.subcore_barrier()
        doff = pl.multiple_of(c * NP + s * RPT, 8)
        pltpu.sync_copy(acc.at[pl.ds(zoff, RPT)], out_hbm.at[pl.ds(doff, RPT)])

    return k(col, zeros16, ones16)


def _scatter_sc(h, row, col, zerosD):
    @functools.partial(
        pl.kernel, mesh=_mesh(),
        out_type=jax.ShapeDtypeStruct((NC * NP, D), jnp.float32),
        scratch_types=[
            pltpu.VMEM((CH,), jnp.int32),
            pltpu.VMEM((CH,), jnp.int32),
            pltpu.VMEM((CH, D), jnp.float32),
            pltpu.VMEM_SHARED((NP, D), jnp.float32),
        ])
    def k(h_hbm, row_hbm, col_hbm, z_hbm, out_hbm, ridx, cidx, rows, acc):
        c = lax.axis_index("c")
        s = lax.axis_index("s")
        zoff = pl.multiple_of(s * RPT, 8)
        pltpu.sync_copy(z_hbm.at[pl.ds(zoff, RPT)], acc.at[pl.ds(zoff, RPT)])
        plsc.subcore_barrier()
        base = (c * NS + s) * EPW

        @pl.loop(0, NCH)
        def _(i):
            off = pl.multiple_of(base + i * CH, 8)
            pltpu.sync_copy(row_hbm.at[pl.ds(off, CH)], ridx)
            pltpu.sync_copy(col_hbm.at[pl.ds(off, CH)], cidx)
            pltpu.sync_copy(h_hbm.at[ridx], rows)
            pltpu.sync_copy(rows, acc.at[cidx], add=True)

        plsc.subcore_barrier()
        doff = pl.multiple_of(c * NP + s * RPT, 8)
        pltpu.sync_copy(acc.at[pl.ds(zoff, RPT)], out_hbm.at[pl.ds(doff, RPT)])

    return k(h, row, col, zerosD)


def _g_of(dp_blk):
    deg = dp_blk[0, :, 0] + dp_blk[1, :, 0]
    return jnp.where(deg > 0, lax.rsqrt(deg), 0.0)


def _mm_scale_tc(x, W, degpair):
    BLK = 1000

    def body(x_ref, w_ref, dp_ref, o_ref):
        g = _g_of(dp_ref)
        h = lax.dot_general(x_ref[...], w_ref[...], (((1,), (1,)), ((), ())),
                            preferred_element_type=jnp.float32)
        o_ref[...] = h * g[:, None]

    return pl.pallas_call(
        body,
        grid=(N // BLK,),
        in_specs=[
            pl.BlockSpec((BLK, D), lambda i: (i, 0)),
            pl.BlockSpec((D, D), lambda i: (0, 0)),
            pl.BlockSpec((2, BLK, DEG_W), lambda i: (0, i, 0)),
        ],
        out_specs=pl.BlockSpec((BLK, D), lambda i: (i, 0)),
        out_shape=jax.ShapeDtypeStruct((N, D), jnp.float32),
    )(x, W, degpair)


def _bn_mm_tc(spair, degpair, b0r, W1):
    def body(sp_ref, dp_ref, b_ref, w_ref, o_ref):
        g = _g_of(dp_ref)
        h = (sp_ref[0] + sp_ref[1]) * g[:, None] + b_ref[...]
        m = jnp.mean(h, axis=0)
        hc = h - m
        v = jnp.mean(hc * hc, axis=0)
        hbn = jnp.maximum(hc * lax.rsqrt(v + 1e-5), 0.0)
        o_ref[...] = lax.dot_general(
            hbn, w_ref[...], (((1,), (1,)), ((), ())),
            preferred_element_type=jnp.float32) * g[:, None]

    return pl.pallas_call(
        body,
        grid=(1,),
        in_specs=[
            pl.BlockSpec((NC, N, D), lambda i: (0, 0, 0)),
            pl.BlockSpec((NC, N, DEG_W), lambda i: (0, 0, 0)),
            pl.BlockSpec((1, D), lambda i: (0, 0)),
            pl.BlockSpec((D, D), lambda i: (0, 0)),
        ],
        out_specs=pl.BlockSpec((N, D), lambda i: (0, 0)),
        out_shape=jax.ShapeDtypeStruct((N, D), jnp.float32),
    )(spair, degpair, b0r, W1)


def _final_tc(tpair, degpair, b1r):
    BLK = 1000

    def body(tp_ref, dp_ref, b_ref, o_ref):
        g = _g_of(dp_ref)
        o_ref[...] = (tp_ref[0] + tp_ref[1]) * g[:, None] + b_ref[...]

    return pl.pallas_call(
        body,
        grid=(N // BLK,),
        in_specs=[
            pl.BlockSpec((2, BLK, D), lambda i: (0, i, 0)),
            pl.BlockSpec((2, BLK, DEG_W), lambda i: (0, i, 0)),
            pl.BlockSpec((1, D), lambda i: (0, 0)),
        ],
        out_specs=pl.BlockSpec((BLK, D), lambda i: (i, 0)),
        out_shape=jax.ShapeDtypeStruct((N, D), jnp.float32),
    )(tpair, degpair, b1r)


def kernel(x, edge_index, W0, b0, W1, b1):
    ei = edge_index.astype(jnp.int32)
    row = ei[0]
    col = ei[1]
    zerosD = jnp.zeros((NP, D), jnp.float32)
    onesD = jnp.ones((CH, DEG_W), jnp.float32)
    b0r = b0.reshape(1, D)
    b1r = b1.reshape(1, D)

    degpair = _deg_sc(col, zerosD, onesD).reshape(NC, NP, DEG_W)
    hs = _mm_scale_tc(x, W0, degpair)
    spair = _scatter_sc(hs, row, col, zerosD).reshape(NC, NP, D)
    hs2 = _bn_mm_tc(spair, degpair, b0r, W1)
    tpair = _scatter_sc(hs2, row, col, zerosD).reshape(NC, NP, D)
    return _final_tc(tpair, degpair, b1r)


# speedup vs baseline: 21.7741x; 2.1298x over previous
import functools

import jax
import jax.numpy as jnp
from jax import lax
from jax.experimental import pallas as pl
from jax.experimental.pallas import tpu as pltpu
from jax.experimental.pallas import tpu_sc as plsc

N = 10000
E = 320000
D = 128
NC = 2
NS = 16
NW = NC * NS
EPW = E // NW
CH = 80
NCH = EPW // CH
NP = 10240
RPT = NP // NS
DEG_W = 128

@functools.cache
def _mesh():
    return plsc.VectorSubcoreMesh(core_axis_name="c", subcore_axis_name="s")


def _deg_sc(col3d, zerosD, ones16):
    @functools.partial(
        pl.kernel, mesh=_mesh(),
        out_type=jax.ShapeDtypeStruct((NC * NP, DEG_W), jnp.float32),
        scratch_types=[
            pltpu.VMEM((NCH, CH), jnp.int32),
            pltpu.VMEM((CH, DEG_W), jnp.float32),
            pltpu.VMEM_SHARED((NP, DEG_W), jnp.float32),
        ])
    def k(col_hbm, z_hbm, ones_hbm, out_hbm, cidx, ones_v, acc):
        c = lax.axis_index("c")
        s = lax.axis_index("s")
        w = c * NS + s
        zoff = pl.multiple_of(s * RPT, 8)
        pltpu.sync_copy(z_hbm.at[pl.ds(zoff, RPT)], acc.at[pl.ds(zoff, RPT)])
        pltpu.sync_copy(ones_hbm, ones_v)
        pltpu.sync_copy(col_hbm.at[w], cidx)
        plsc.subcore_barrier()

        @pl.loop(0, NCH)
        def _(i):
            pltpu.sync_copy(ones_v, acc.at[cidx.at[i]], add=True)

        plsc.subcore_barrier()
        doff = pl.multiple_of(c * NP + s * RPT, 8)
        pltpu.sync_copy(acc.at[pl.ds(zoff, RPT)], out_hbm.at[pl.ds(doff, RPT)])

    return k(col3d, zerosD, ones16)


def _scatter_sc(h, row1d, col3d, zerosD):
    @functools.partial(
        pl.kernel, mesh=_mesh(),
        out_type=jax.ShapeDtypeStruct((NC * NP, D), jnp.float32),
        scratch_types=[
            pltpu.VMEM((EPW,), jnp.int32),
            pltpu.VMEM((NCH, CH), jnp.int32),
            pltpu.VMEM((CH, D), jnp.float32),
            pltpu.VMEM((CH, D), jnp.float32),
            pltpu.VMEM_SHARED((NP, D), jnp.float32),
            pltpu.SemaphoreType.DMA,
            pltpu.SemaphoreType.DMA,
        ])
    def k(h_hbm, row_hbm, col_hbm, z_hbm, out_hbm, ridx, cidx, rows_a,
          rows_b, acc, sem_a, sem_b):
        c = lax.axis_index("c")
        s = lax.axis_index("s")
        w = c * NS + s
        zoff = pl.multiple_of(s * RPT, 8)
        pltpu.sync_copy(z_hbm.at[pl.ds(zoff, RPT)], acc.at[pl.ds(zoff, RPT)])
        pltpu.sync_copy(row_hbm.at[pl.ds(w * EPW, EPW)], ridx)
        pltpu.sync_copy(col_hbm.at[w], cidx)
        plsc.subcore_barrier()

        def gat(i, buf, sem):
            off = pl.multiple_of(i * CH, 8)
            pltpu.async_copy(h_hbm.at[ridx.at[pl.ds(off, CH)]], buf, sem)

        def gwait(i, buf, sem):
            off = pl.multiple_of(i * CH, 8)
            pltpu.make_async_copy(h_hbm.at[ridx.at[pl.ds(off, CH)]], buf,
                                  sem).wait()

        gat(0, rows_a, sem_a)

        @pl.loop(0, NCH // 2)
        def _(j):
            i = j * 2
            gat(i + 1, rows_b, sem_b)
            gwait(i, rows_a, sem_a)
            pltpu.sync_copy(rows_a, acc.at[cidx.at[i]], add=True)
            gat(i + 2, rows_a, sem_a)
            gwait(i + 1, rows_b, sem_b)
            pltpu.sync_copy(rows_b, acc.at[cidx.at[i + 1]], add=True)

        gwait(NCH - 1, rows_a, sem_a)
        pltpu.sync_copy(rows_a, acc.at[cidx.at[NCH - 1]], add=True)

        plsc.subcore_barrier()
        doff = pl.multiple_of(c * NP + s * RPT, 8)
        pltpu.sync_copy(acc.at[pl.ds(zoff, RPT)], out_hbm.at[pl.ds(doff, RPT)])

    return k(h, row1d, col3d, zerosD)


def _g_of(dp_blk):
    deg = dp_blk[0, :, 0] + dp_blk[1, :, 0]
    return jnp.where(deg > 0, lax.rsqrt(deg), 0.0)


def _mm_scale_tc(x, W, degpair):
    BLK = 1000

    def body(x_ref, w_ref, dp_ref, o_ref):
        g = _g_of(dp_ref)
        h = lax.dot_general(x_ref[...], w_ref[...], (((1,), (1,)), ((), ())),
                            preferred_element_type=jnp.float32)
        o_ref[...] = h * g[:, None]

    return pl.pallas_call(
        body,
        grid=(N // BLK,),
        in_specs=[
            pl.BlockSpec((BLK, D), lambda i: (i, 0)),
            pl.BlockSpec((D, D), lambda i: (0, 0)),
            pl.BlockSpec((2, BLK, DEG_W), lambda i: (0, i, 0)),
        ],
        out_specs=pl.BlockSpec((BLK, D), lambda i: (i, 0)),
        out_shape=jax.ShapeDtypeStruct((N, D), jnp.float32),
    )(x, W, degpair)


def _bn_mm_tc(spair, degpair, b0r, W1):
    def body(sp_ref, dp_ref, b_ref, w_ref, o_ref):
        g = _g_of(dp_ref)
        h = (sp_ref[0] + sp_ref[1]) * g[:, None] + b_ref[...]
        m = jnp.mean(h, axis=0)
        hc = h - m
        v = jnp.mean(hc * hc, axis=0)
        hbn = jnp.maximum(hc * lax.rsqrt(v + 1e-5), 0.0)
        o_ref[...] = lax.dot_general(
            hbn, w_ref[...], (((1,), (1,)), ((), ())),
            preferred_element_type=jnp.float32) * g[:, None]

    return pl.pallas_call(
        body,
        grid=(1,),
        in_specs=[
            pl.BlockSpec((NC, N, D), lambda i: (0, 0, 0)),
            pl.BlockSpec((NC, N, DEG_W), lambda i: (0, 0, 0)),
            pl.BlockSpec((1, D), lambda i: (0, 0)),
            pl.BlockSpec((D, D), lambda i: (0, 0)),
        ],
        out_specs=pl.BlockSpec((N, D), lambda i: (0, 0)),
        out_shape=jax.ShapeDtypeStruct((N, D), jnp.float32),
    )(spair, degpair, b0r, W1)


def _final_tc(tpair, degpair, b1r):
    BLK = 1000

    def body(tp_ref, dp_ref, b_ref, o_ref):
        g = _g_of(dp_ref)
        o_ref[...] = (tp_ref[0] + tp_ref[1]) * g[:, None] + b_ref[...]

    return pl.pallas_call(
        body,
        grid=(N // BLK,),
        in_specs=[
            pl.BlockSpec((2, BLK, D), lambda i: (0, i, 0)),
            pl.BlockSpec((2, BLK, DEG_W), lambda i: (0, i, 0)),
            pl.BlockSpec((1, D), lambda i: (0, 0)),
        ],
        out_specs=pl.BlockSpec((BLK, D), lambda i: (i, 0)),
        out_shape=jax.ShapeDtypeStruct((N, D), jnp.float32),
    )(tpair, degpair, b1r)


def kernel(x, edge_index, W0, b0, W1, b1):
    ei = edge_index.astype(jnp.int32)
    row1d = ei[0]
    col3d = ei[1].reshape(NW, NCH, CH)
    zerosD = jnp.zeros((NP, D), jnp.float32)
    onesD = jnp.ones((CH, DEG_W), jnp.float32)
    b0r = b0.reshape(1, D)
    b1r = b1.reshape(1, D)

    degpair = _deg_sc(col3d, zerosD, onesD).reshape(NC, NP, DEG_W)
    hs = _mm_scale_tc(x, W0, degpair)
    spair = _scatter_sc(hs, row1d, col3d, zerosD).reshape(NC, NP, D)
    hs2 = _bn_mm_tc(spair, degpair, b0r, W1)
    tpair = _scatter_sc(hs2, row1d, col3d, zerosD).reshape(NC, NP, D)
    return _final_tc(tpair, degpair, b1r)
